# baseline (speedup 1.0000x reference)
import functools

import jax
import jax.numpy as jnp
from jax import lax
from jax.experimental import pallas as pl
from jax.experimental.pallas import tpu as pltpu
from jax.experimental.pallas import tpu_sc as plsc

N_NODES = 10000
N_EDGES = 320000
D_FEAT = 128

_NC = 2
_NS = 16
_NW = _NC * _NS
_E_PER_W = N_EDGES // _NW
_LANES = 16
_UNROLL = 8


_TC_CHUNK = 2048
_TC_CHUNKS = [(i * _TC_CHUNK, _TC_CHUNK) for i in range(N_NODES // _TC_CHUNK)]
_TC_REM = N_NODES - len(_TC_CHUNKS) * _TC_CHUNK
if _TC_REM:
    _TC_CHUNKS.append((len(_TC_CHUNKS) * _TC_CHUNK, _TC_REM))


def _matvec_body(x_hbm, w_ref, b_ref, t_ref, xb, *sems):
    copies = []
    for k, (off, size) in enumerate(_TC_CHUNKS):
        c = pltpu.make_async_copy(
            x_hbm.at[pl.ds(off, size)], xb.at[k].at[pl.ds(0, size)], sems[k])
        c.start()
        copies.append(c)
    for k, (off, size) in enumerate(_TC_CHUNKS):
        copies[k].wait()
        out = lax.dot_general(
            w_ref[...], xb[k, pl.ds(0, size), :],
            (((1,), (1,)), ((), ())),
            preferred_element_type=jnp.float32,
        )
        p = (out[0] + b_ref[0]).astype(jnp.bfloat16)
        q = out[1].astype(jnp.bfloat16)
        p_bits = lax.bitcast_convert_type(p, jnp.uint16).astype(jnp.uint32)
        q_bits = lax.bitcast_convert_type(q, jnp.uint16).astype(jnp.uint32)
        t_ref[pl.ds(off, size)] = ((p_bits << 16) | q_bits).astype(jnp.int32)


def _node_tables(x, W, b):
    w2 = W.reshape(2, D_FEAT)
    return pl.pallas_call(
        _matvec_body,
        in_specs=[
            pl.BlockSpec(memory_space=pl.ANY),
            pl.BlockSpec(memory_space=pltpu.VMEM),
            pl.BlockSpec(memory_space=pltpu.SMEM),
        ],
        scratch_shapes=[
            pltpu.VMEM((len(_TC_CHUNKS), _TC_CHUNK, D_FEAT), jnp.float32),
        ] + [pltpu.SemaphoreType.DMA] * len(_TC_CHUNKS),
        out_shape=jax.ShapeDtypeStruct((N_NODES,), jnp.int32),
    )(x, w2, b)


def _make_sc_kernel():
    mesh = plsc.VectorSubcoreMesh(core_axis_name="c", subcore_axis_name="s")

    @functools.partial(
        pl.kernel,
        mesh=mesh,
        out_type=jax.ShapeDtypeStruct((N_EDGES,), jnp.float32),
        compiler_params=pltpu.CompilerParams(needs_layout_passes=False),
        scratch_types=[
            pltpu.VMEM((N_NODES,), jnp.int32),
            pltpu.VMEM((_E_PER_W,), jnp.int32),
            pltpu.VMEM((_E_PER_W,), jnp.int32),
            pltpu.VMEM((_E_PER_W,), jnp.float32),
            pltpu.SemaphoreType.DMA,
        ],
    )
    def sc_edge_kernel(t_hbm, src_hbm, trg_hbm, out_hbm,
                       t_v, src_v, trg_v, out_v, sem):
        wid = lax.axis_index("s") * _NC + lax.axis_index("c")
        base = wid * _E_PER_W
        c1 = pltpu.async_copy(t_hbm, t_v, sem)
        c2 = pltpu.async_copy(src_hbm.at[pl.ds(base, _E_PER_W)], src_v, sem)
        c3 = pltpu.async_copy(trg_hbm.at[pl.ds(base, _E_PER_W)], trg_v, sem)
        c1.wait()
        c2.wait()
        c3.wait()

        hi_mask = jnp.int32(-65536)

        @plsc.parallel_loop(0, _E_PER_W // _LANES, 1, unroll=_UNROLL)
        def _body(i):
            off = i * _LANES
            si = src_v[pl.ds(off, _LANES)]
            ti = trg_v[pl.ds(off, _LANES)]
            sw = plsc.load_gather(t_v, [si])
            tw = plsc.load_gather(t_v, [ti])
            pv = plsc.bitcast(sw & hi_mask, jnp.float32)
            qv = plsc.bitcast(tw << 16, jnp.float32)
            z = pv + qv
            out_v[pl.ds(off, _LANES)] = 1.0 / (1.0 + jnp.exp(-z))

        pltpu.sync_copy(out_v, out_hbm.at[pl.ds(base, _E_PER_W)])

    return sc_edge_kernel


_SC_KERNEL = _make_sc_kernel()


def kernel(input, edge_src_nodes, edge_trg_nodes, W, b):
    x = input.reshape(-1, input.shape[-1]).astype(jnp.float32)
    t = _node_tables(x, W.astype(jnp.float32), b.astype(jnp.float32))
    src = edge_src_nodes.astype(jnp.int32)
    trg = edge_trg_nodes.astype(jnp.int32)
    return _SC_KERNEL(t, src, trg).reshape(N_EDGES, 1)


# speedup vs baseline: 1.0077x; 1.0077x over previous
import functools

import jax
import jax.numpy as jnp
from jax import lax
from jax.experimental import pallas as pl
from jax.experimental.pallas import tpu as pltpu
from jax.experimental.pallas import tpu_sc as plsc

N_NODES = 10000
N_EDGES = 320000
D_FEAT = 128

_NC = 2
_NS = 16
_NW = _NC * _NS
_E_PER_W = N_EDGES // _NW
_LANES = 16
_UNROLL = 8


def _matvec_body(x_ref, w_ref, b_ref, t_ref):
    out = lax.dot_general(
        w_ref[...], x_ref[...],
        (((1,), (1,)), ((), ())),
        preferred_element_type=jnp.float32,
    )
    p = (out[0] + b_ref[0]).astype(jnp.bfloat16)
    q = out[1].astype(jnp.bfloat16)
    p_bits = lax.bitcast_convert_type(p, jnp.uint16).astype(jnp.uint32)
    q_bits = lax.bitcast_convert_type(q, jnp.uint16).astype(jnp.uint32)
    t_ref[...] = ((p_bits << 16) | q_bits).astype(jnp.int32)


def _node_tables(x, W, b):
    w2 = W.reshape(2, D_FEAT)
    return pl.pallas_call(
        _matvec_body,
        in_specs=[
            pl.BlockSpec(memory_space=pltpu.VMEM),
            pl.BlockSpec(memory_space=pltpu.VMEM),
            pl.BlockSpec(memory_space=pltpu.SMEM),
        ],
        out_shape=jax.ShapeDtypeStruct((N_NODES,), jnp.int32),
    )(x, w2, b)


def _make_sc_kernel():
    mesh = plsc.VectorSubcoreMesh(core_axis_name="c", subcore_axis_name="s")

    @functools.partial(
        pl.kernel,
        mesh=mesh,
        out_type=jax.ShapeDtypeStruct((N_EDGES,), jnp.float32),
        compiler_params=pltpu.CompilerParams(needs_layout_passes=False),
        scratch_types=[
            pltpu.VMEM((N_NODES,), jnp.int32),
            pltpu.VMEM((_E_PER_W,), jnp.int32),
            pltpu.VMEM((_E_PER_W,), jnp.int32),
            pltpu.VMEM((_E_PER_W,), jnp.float32),
            pltpu.SemaphoreType.DMA,
        ],
    )
    def sc_edge_kernel(t_hbm, src_hbm, trg_hbm, out_hbm,
                       t_v, src_v, trg_v, out_v, sem):
        wid = lax.axis_index("s") * _NC + lax.axis_index("c")
        base = wid * _E_PER_W
        c1 = pltpu.async_copy(t_hbm, t_v, sem)
        c2 = pltpu.async_copy(src_hbm.at[pl.ds(base, _E_PER_W)], src_v, sem)
        c3 = pltpu.async_copy(trg_hbm.at[pl.ds(base, _E_PER_W)], trg_v, sem)
        c1.wait()
        c2.wait()
        c3.wait()

        hi_mask = jnp.int32(-65536)

        @plsc.parallel_loop(0, _E_PER_W // _LANES, 1, unroll=_UNROLL)
        def _body(i):
            off = i * _LANES
            si = src_v[pl.ds(off, _LANES)]
            ti = trg_v[pl.ds(off, _LANES)]
            sw = plsc.load_gather(t_v, [si])
            tw = plsc.load_gather(t_v, [ti])
            pv = plsc.bitcast(sw & hi_mask, jnp.float32)
            qv = plsc.bitcast(tw << 16, jnp.float32)
            z = pv + qv
            out_v[pl.ds(off, _LANES)] = 1.0 / (1.0 + jnp.exp(-z))

        pltpu.sync_copy(out_v, out_hbm.at[pl.ds(base, _E_PER_W)])

    return sc_edge_kernel


_SC_KERNEL = _make_sc_kernel()


def kernel(input, edge_src_nodes, edge_trg_nodes, W, b):
    x = input.reshape(-1, input.shape[-1]).astype(jnp.float32)
    t = _node_tables(x, W.astype(jnp.float32), b.astype(jnp.float32))
    src = edge_src_nodes.astype(jnp.int32)
    trg = edge_trg_nodes.astype(jnp.int32)
    return _SC_KERNEL(t, src, trg).reshape(N_EDGES, 1)
